# Initial kernel scaffold; baseline (speedup 1.0000x reference)
import jax
import jax.numpy as jnp
from jax.experimental import pallas as pl


def kernel(xyz, feats, W1, g1, b1, rm1, rv1, W2, g2, b2, rm2, rv2, W3, b3):
    raise NotImplementedError("write your pallas kernel here")


# speedup vs baseline: 10.9641x; 10.9641x over previous
import functools

import jax
import jax.numpy as jnp
from jax.experimental import pallas as pl

_B, _N, _C, _H, _M = 4, 16384, 128, 128, 2048
_EPS = 1e-5
_GAMMA = 1.0
_NR, _NC = 128, 128
_BLK = 2048


def _mlp_kernel(feats_ref, w1_ref, g1_ref, b1_ref, rm1_ref, t1_ref,
                w2_ref, g2_ref, b2_ref, rm2_ref, t2_ref, w3p_ref, b3_ref,
                out_ref):
    x = feats_ref[...]
    h = jnp.dot(x, w1_ref[...], preferred_element_type=jnp.float32)
    h = (h - rm1_ref[...]) / t1_ref[...] * g1_ref[...] + b1_ref[...]
    h = jnp.maximum(h, 0.0)
    h = jnp.dot(h, w2_ref[...], preferred_element_type=jnp.float32)
    h = (h - rm2_ref[...]) / t2_ref[...] * g2_ref[...] + b2_ref[...]
    h = jnp.maximum(h, 0.0)
    s = jnp.dot(h, w3p_ref[...], preferred_element_type=jnp.float32) + b3_ref[...]
    out_ref[...] = s


def _mlp_scores_call(feats2, W1, g1, b1, rm1, rv1, W2, g2, b2, rm2, rv2, W3,
                     b3, interpret=False):
    rows = feats2.shape[0]
    grid = rows // _BLK
    w3p = jnp.pad(W3, ((0, 0), (0, _NC - W3.shape[1])))
    t1 = jnp.sqrt(rv1 + _EPS)
    t2 = jnp.sqrt(rv2 + _EPS)
    vec = lambda v: v.reshape(1, -1)
    full = pl.BlockSpec((_BLK, _C), lambda i: (i, 0))
    wspec = pl.BlockSpec((_C, _H), lambda i: (0, 0))
    vspec = pl.BlockSpec((1, _H), lambda i: (0, 0))
    out = pl.pallas_call(
        _mlp_kernel,
        grid=(grid,),
        in_specs=[full, wspec, vspec, vspec, vspec, vspec,
                  wspec, vspec, vspec, vspec, vspec,
                  wspec, pl.BlockSpec((1, 1), lambda i: (0, 0))],
        out_specs=pl.BlockSpec((_BLK, _NC), lambda i: (i, 0)),
        out_shape=jax.ShapeDtypeStruct((rows, _NC), jnp.float32),
        interpret=interpret,
    )(feats2, W1, vec(g1), vec(b1), vec(rm1), vec(t1),
      W2, vec(g2), vec(b2), vec(rm2), vec(t2), w3p, b3.reshape(1, 1))
    return out[:, 0]


def _fps_kernel(m, xyz_ref, w_ref, out_ref):
    idx2d = (jax.lax.broadcasted_iota(jnp.int32, (_NR, _NC), 0) * _NC
             + jax.lax.broadcasted_iota(jnp.int32, (_NR, _NC), 1))
    lane = jax.lax.broadcasted_iota(jnp.int32, (1, _NC), 1)

    def body(step, carry):
        temps, fars = carry
        new_temps, new_fars = [], []
        for b in range(_B):
            far = fars[b]
            out_ref[b, pl.ds(step, 1), :] = far.reshape(1, 1)
            r = far // _NC
            c = far % _NC
            sel = lane == c
            xf = jnp.sum(jnp.where(sel, xyz_ref[b, 0, pl.ds(r, 1), :], 0.0))
            yf = jnp.sum(jnp.where(sel, xyz_ref[b, 1, pl.ds(r, 1), :], 0.0))
            zf = jnp.sum(jnp.where(sel, xyz_ref[b, 2, pl.ds(r, 1), :], 0.0))
            dx = xyz_ref[b, 0] - xf
            dy = xyz_ref[b, 1] - yf
            dz = xyz_ref[b, 2] - zf
            d = ((dx * dx + dy * dy) + dz * dz) * w_ref[b]
            t = jnp.minimum(temps[b], d)
            mx = jnp.max(t)
            nxt = jnp.min(jnp.where(t == mx, idx2d, jnp.int32(_N)))
            new_temps.append(t)
            new_fars.append(nxt)
        return tuple(new_temps), tuple(new_fars)

    init = (tuple(jnp.full((_NR, _NC), 1e10, dtype=jnp.float32)
                  for _ in range(_B)),
            tuple(jnp.int32(0) for _ in range(_B)))
    jax.lax.fori_loop(0, m, body, init)


def _fps_call(xyzp, wts, m, interpret=False):
    out = pl.pallas_call(
        functools.partial(_fps_kernel, m),
        out_shape=jax.ShapeDtypeStruct((_B, m, 1), jnp.int32),
        interpret=interpret,
    )(xyzp, wts)
    return out.reshape(_B, m)


def kernel(xyz, feats, W1, g1, b1, rm1, rv1, W2, g2, b2, rm2, rv2, W3, b3):
    feats2 = feats.reshape(_B * _N, _C)
    scores = _mlp_scores_call(feats2, W1, g1, b1, rm1, rv1,
                              W2, g2, b2, rm2, rv2, W3, b3).reshape(_B, _N)
    weights = jax.nn.sigmoid(scores) ** _GAMMA
    xyzp = xyz.transpose(0, 2, 1).reshape(_B, 3, _NR, _NC)
    wts = weights.reshape(_B, _NR, _NC)
    idx = _fps_call(xyzp, wts, _M)
    return idx.astype(jnp.int64)


# speedup vs baseline: 27.3780x; 2.4971x over previous
import functools

import jax
import jax.numpy as jnp
from jax.experimental import pallas as pl

_B, _N, _C, _H, _M = 4, 16384, 128, 128, 2048
_EPS = 1e-5
_GAMMA = 1.0
_NR, _NC = 128, 128
_BLK = 2048


def _mlp_kernel(feats_ref, w1_ref, g1_ref, b1_ref, rm1_ref, t1_ref,
                w2_ref, g2_ref, b2_ref, rm2_ref, t2_ref, w3p_ref, b3_ref,
                out_ref):
    x = feats_ref[...]
    h = jnp.dot(x, w1_ref[...], preferred_element_type=jnp.float32)
    h = (h - rm1_ref[...]) / t1_ref[...] * g1_ref[...] + b1_ref[...]
    h = jnp.maximum(h, 0.0)
    h = jnp.dot(h, w2_ref[...], preferred_element_type=jnp.float32)
    h = (h - rm2_ref[...]) / t2_ref[...] * g2_ref[...] + b2_ref[...]
    h = jnp.maximum(h, 0.0)
    s = jnp.dot(h, w3p_ref[...], preferred_element_type=jnp.float32) + b3_ref[...]
    out_ref[...] = s


def _mlp_scores_call(feats2, W1, g1, b1, rm1, rv1, W2, g2, b2, rm2, rv2, W3,
                     b3, interpret=False):
    rows = feats2.shape[0]
    grid = rows // _BLK
    w3p = jnp.pad(W3, ((0, 0), (0, _NC - W3.shape[1])))
    t1 = jnp.sqrt(rv1 + _EPS)
    t2 = jnp.sqrt(rv2 + _EPS)
    vec = lambda v: v.reshape(1, -1)
    full = pl.BlockSpec((_BLK, _C), lambda i: (i, 0))
    wspec = pl.BlockSpec((_C, _H), lambda i: (0, 0))
    vspec = pl.BlockSpec((1, _H), lambda i: (0, 0))
    out = pl.pallas_call(
        _mlp_kernel,
        grid=(grid,),
        in_specs=[full, wspec, vspec, vspec, vspec, vspec,
                  wspec, vspec, vspec, vspec, vspec,
                  wspec, pl.BlockSpec((1, 1), lambda i: (0, 0))],
        out_specs=pl.BlockSpec((_BLK, _NC), lambda i: (i, 0)),
        out_shape=jax.ShapeDtypeStruct((rows, _NC), jnp.float32),
        interpret=interpret,
    )(feats2, W1, vec(g1), vec(b1), vec(rm1), vec(t1),
      W2, vec(g2), vec(b2), vec(rm2), vec(t2), w3p, b3.reshape(1, 1))
    return out[:, 0]


def _fps_kernel(m, xyz_ref, w_ref, out_ref):
    mrows = (m + _NC - 1) // _NC
    idx2d = (jax.lax.broadcasted_iota(jnp.int32, (_NR, _NC), 0) * _NC
             + jax.lax.broadcasted_iota(jnp.int32, (_NR, _NC), 1))
    step2d = (jax.lax.broadcasted_iota(jnp.int32, (mrows, _NC), 0) * _NC
              + jax.lax.broadcasted_iota(jnp.int32, (mrows, _NC), 1))
    lane = jax.lax.broadcasted_iota(jnp.int32, (1, _NC), 1)

    def body(step, carry):
        temps, fars, accs = carry
        onehot = step2d == step
        new_accs = [jnp.where(onehot, fars[b], accs[b]) for b in range(_B)]
        masks = [idx2d == fars[b] for b in range(_B)]
        coords = [[jnp.sum(jnp.where(masks[b], xyz_ref[b, k], 0.0))
                   for k in range(3)] for b in range(_B)]
        ts = []
        for b in range(_B):
            dx = xyz_ref[b, 0] - coords[b][0]
            dy = xyz_ref[b, 1] - coords[b][1]
            dz = xyz_ref[b, 2] - coords[b][2]
            d = ((dx * dx + dy * dy) + dz * dz) * w_ref[b]
            ts.append(jnp.minimum(temps[b], d))
        mxs = [jnp.max(ts[b]) for b in range(_B)]
        nxts = [jnp.min(jnp.where(ts[b] == mxs[b], idx2d, jnp.int32(_N)))
                for b in range(_B)]
        return tuple(ts), tuple(nxts), tuple(new_accs)

    init = (tuple(jnp.full((_NR, _NC), 1e10, dtype=jnp.float32)
                  for _ in range(_B)),
            tuple(jnp.int32(0) for _ in range(_B)),
            tuple(jnp.zeros((mrows, _NC), dtype=jnp.int32) for _ in range(_B)))
    _, _, accs = jax.lax.fori_loop(0, m, body, init)
    for b in range(_B):
        out_ref[b] = accs[b]


def _fps_call(xyzp, wts, m, interpret=False):
    mrows = (m + _NC - 1) // _NC
    out = pl.pallas_call(
        functools.partial(_fps_kernel, m),
        out_shape=jax.ShapeDtypeStruct((_B, mrows, _NC), jnp.int32),
        interpret=interpret,
    )(xyzp, wts)
    return out.reshape(_B, mrows * _NC)[:, :m]


def kernel(xyz, feats, W1, g1, b1, rm1, rv1, W2, g2, b2, rm2, rv2, W3, b3):
    feats2 = feats.reshape(_B * _N, _C)
    scores = _mlp_scores_call(feats2, W1, g1, b1, rm1, rv1,
                              W2, g2, b2, rm2, rv2, W3, b3).reshape(_B, _N)
    weights = jax.nn.sigmoid(scores) ** _GAMMA
    xyzp = xyz.transpose(0, 2, 1).reshape(_B, 3, _NR, _NC)
    wts = weights.reshape(_B, _NR, _NC)
    idx = _fps_call(xyzp, wts, _M)
    return idx.astype(jnp.int64)
